# Initial kernel scaffold; baseline (speedup 1.0000x reference)
import jax
import jax.numpy as jnp
from jax.experimental import pallas as pl


def kernel(x, edge_index, edge_type, batch, We, be, W1, W1_root, b1, W2, W2_root, b2, fc1_W, fc1_b, fc2_W, fc2_b):
    raise NotImplementedError("write your pallas kernel here")


# speedup vs baseline: 15.2725x; 15.2725x over previous
import functools
import jax
import jax.numpy as jnp
from jax import lax
from jax.experimental import pallas as pl
from jax.experimental.pallas import tpu as pltpu
from jax.experimental.pallas import tpu_sc as plsc

N = 10000
E = 320000
R = 8
NT = 8
F = 128
H1 = 16
H2 = 32
NG = 64
NC = 10

NCORE = 2
NSUB = 16
NW = NCORE * NSUB
LANE = 128
RPT = (E + NW * LANE - 1) // (NW * LANE)
E_PAD = NW * LANE * RPT
NROWS = N * R
DUMMY = NROWS
ZCH = 5008
NROWS_PAD = ZCH * NSUB

BN = 1000
NB = N // BN


def _flat_idx_body(src_ref, dst_ref, typ_ref, sflat_ref, dflat_ref):
    t = typ_ref[...]
    sflat_ref[...] = src_ref[...] * R + t
    dflat_ref[...] = dst_ref[...] * R + t


def _dense1_body(x_ref, WeT_ref, be_ref, W1cat_ref, W1root_ref, b1_ref,
                 ha_ref, r1_ref):
    h = jnp.dot(x_ref[...], WeT_ref[...], preferred_element_type=jnp.float32)
    h = h + be_ref[...]
    ha_ref[...] = jnp.dot(h, W1cat_ref[...], preferred_element_type=jnp.float32)
    r1_ref[...] = jnp.dot(h, W1root_ref[...],
                          preferred_element_type=jnp.float32) + b1_ref[...]


def _dense2_body(c0_ref, c1_ref, p0_ref, p1_ref, r1_ref, S_ref,
                 W2a_ref, W2b_ref, W2root_ref, b2_ref,
                 h2a_ref, h2b_ref, r2_ref, cc_ref):
    cc = jnp.maximum(c0_ref[...] + c1_ref[...], 1.0)
    t = (p0_ref[...] + p1_ref[...]) / cc
    agg = jnp.dot(t, S_ref[...], preferred_element_type=jnp.float32)
    o1 = jnp.maximum(r1_ref[...] + agg, 0.0)
    h2a_ref[...] = jnp.dot(o1, W2a_ref[...], preferred_element_type=jnp.float32)
    h2b_ref[...] = jnp.dot(o1, W2b_ref[...], preferred_element_type=jnp.float32)
    r2_ref[...] = jnp.dot(o1, W2root_ref[...],
                          preferred_element_type=jnp.float32) + b2_ref[...]
    cc_ref[...] = cc


def _combine_pool_body(qa0_ref, qa1_ref, qb0_ref, qb1_ref, cc_ref, r2_ref,
                       batch_ref, S_ref, sa_ref, sb_ref, cg_ref):
    cc = cc_ref[...]
    ta = jnp.dot((qa0_ref[...] + qa1_ref[...]) / cc, S_ref[...],
                 preferred_element_type=jnp.float32)
    tb = jnp.dot((qb0_ref[...] + qb1_ref[...]) / cc, S_ref[...],
                 preferred_element_type=jnp.float32)
    r2 = r2_ref[...]
    o2a = jnp.maximum(r2[:, :H1] + ta, 0.0)
    o2b = jnp.maximum(r2[:, H1:] + tb, 0.0)
    brow = batch_ref[0]
    gids = lax.broadcasted_iota(jnp.int32, (NG, BN), 0)
    oh = (gids == jnp.broadcast_to(brow, (NG, BN))).astype(jnp.float32)
    sa_c = jnp.dot(oh, o2a, preferred_element_type=jnp.float32)
    sb_c = jnp.dot(oh, o2b, preferred_element_type=jnp.float32)
    cg_c = jnp.broadcast_to(jnp.sum(oh, axis=1, keepdims=True), (NG, H1))

    @pl.when(pl.program_id(0) == 0)
    def _init():
        sa_ref[...] = jnp.zeros_like(sa_ref)
        sb_ref[...] = jnp.zeros_like(sb_ref)
        cg_ref[...] = jnp.zeros_like(cg_ref)

    sa_ref[...] += sa_c
    sb_ref[...] += sb_c
    cg_ref[...] += cg_c


def _head_body(sa_ref, sb_ref, cg_ref, fc1aT_ref, fc1bT_ref, fc1b_ref,
               fc2T_ref, fc2b_ref, out_ref):
    c = jnp.maximum(cg_ref[...], 1.0)
    pa = sa_ref[...] / c
    pb = sb_ref[...] / c
    hh = jnp.dot(pa, fc1aT_ref[...], preferred_element_type=jnp.float32)
    hh = hh + jnp.dot(pb, fc1bT_ref[...], preferred_element_type=jnp.float32)
    hh = jnp.maximum(hh + fc1b_ref[...], 0.0)
    out_ref[...] = jnp.dot(hh, fc2T_ref[...],
                           preferred_element_type=jnp.float32) + fc2b_ref[...]


_SC_MESH = plsc.VectorSubcoreMesh(core_axis_name="c", subcore_axis_name="s")


def _sc_count(dflat_hbm, ones_hbm, zeros_hbm, out_hbm, dst_v, ones_v, acc):
    c = lax.axis_index("c")
    s = lax.axis_index("s")
    wid = c * NSUB + s
    pltpu.sync_copy(zeros_hbm.at[pl.ds(s * ZCH, ZCH)],
                    acc.at[pl.ds(s * ZCH, ZCH)])
    pltpu.sync_copy(ones_hbm, ones_v)
    pltpu.sync_copy(dflat_hbm.at[wid], dst_v)
    plsc.subcore_barrier()

    def step(j, carry):
        pltpu.sync_copy(ones_v, acc.at[dst_v.at[j]], add=True)
        return carry

    lax.fori_loop(0, RPT, step, 0)
    plsc.subcore_barrier()
    pltpu.sync_copy(acc.at[pl.ds(s * ZCH, ZCH)],
                    out_hbm.at[c, pl.ds(s * ZCH, ZCH)])


def _sc_edge(table_hbm, sflat_hbm, dflat_hbm, zeros_hbm, out_hbm,
             src_v, dst_v, msg_v, acc, sem):
    c = lax.axis_index("c")
    s = lax.axis_index("s")
    wid = c * NSUB + s
    pltpu.sync_copy(zeros_hbm.at[pl.ds(s * ZCH, ZCH)],
                    acc.at[pl.ds(s * ZCH, ZCH)])
    pltpu.sync_copy(sflat_hbm.at[wid], src_v)
    pltpu.sync_copy(dflat_hbm.at[wid], dst_v)
    plsc.subcore_barrier()

    def step(j, carry):
        pltpu.async_copy(table_hbm.at[src_v.at[j]], msg_v, sem).wait()
        pltpu.sync_copy(msg_v, acc.at[dst_v.at[j]], add=True)
        return carry

    lax.fori_loop(0, RPT, step, 0)
    plsc.subcore_barrier()
    pltpu.sync_copy(acc.at[pl.ds(s * ZCH, ZCH)],
                    out_hbm.at[c, pl.ds(s * ZCH, ZCH)])


_SC_PARAMS = pltpu.CompilerParams(use_tc_tiling_on_sc=False)

_sc_count_call = pl.kernel(
    _sc_count,
    out_type=jax.ShapeDtypeStruct((NCORE, NROWS_PAD, H1), jnp.float32),
    mesh=_SC_MESH,
    compiler_params=_SC_PARAMS,
    scratch_types=[
        pltpu.VMEM((RPT, LANE), jnp.int32),
        pltpu.VMEM((LANE, H1), jnp.float32),
        pltpu.VMEM_SHARED((NROWS_PAD, H1), jnp.float32),
    ],
)

_sc_edge_call = pl.kernel(
    _sc_edge,
    out_type=jax.ShapeDtypeStruct((NCORE, NROWS_PAD, H1), jnp.float32),
    mesh=_SC_MESH,
    compiler_params=_SC_PARAMS,
    scratch_types=[
        pltpu.VMEM((RPT, LANE), jnp.int32),
        pltpu.VMEM((RPT, LANE), jnp.int32),
        pltpu.VMEM((LANE, H1), jnp.float32),
        pltpu.VMEM_SHARED((NROWS_PAD, H1), jnp.float32),
        pltpu.SemaphoreType.DMA,
    ],
)


def kernel(x, edge_index, edge_type, batch, We, be, W1, W1_root, b1,
           W2, W2_root, b2, fc1_W, fc1_b, fc2_W, fc2_b):
    f32 = jnp.float32
    pad = E_PAD - E
    src_p = jnp.pad(edge_index[0].astype(jnp.int32), (0, pad)).reshape(-1, LANE)
    dst_p = jnp.pad(edge_index[1].astype(jnp.int32), (0, pad),
                    constant_values=N).reshape(-1, LANE)
    typ_p = jnp.pad(edge_type.astype(jnp.int32), (0, pad)).reshape(-1, LANE)

    nrows2d = E_PAD // LANE
    sflat, dflat = pl.pallas_call(
        _flat_idx_body,
        out_shape=(jax.ShapeDtypeStruct((nrows2d, LANE), jnp.int32),
                   jax.ShapeDtypeStruct((nrows2d, LANE), jnp.int32)),
    )(src_p, dst_p, typ_p)
    sflat3 = sflat.reshape(NW, RPT, LANE)
    dflat3 = dflat.reshape(NW, RPT, LANE)

    zeros_acc = jnp.zeros((NROWS_PAD, H1), f32)
    ones_rows = jnp.ones((LANE, H1), f32)

    cnt_part = _sc_count_call(dflat3, ones_rows, zeros_acc)
    cexp = cnt_part[:, :NROWS].reshape(NCORE, N, R * H1)

    W1cat = jnp.transpose(W1, (1, 0, 2)).reshape(F, R * H1)
    row_spec = lambda w: pl.BlockSpec((BN, w), lambda i: (i, 0))
    full = lambda a: pl.BlockSpec(a.shape, lambda i: (0,) * a.ndim)
    WeT = We.T
    be_r = be.reshape(1, F)
    b1_r = b1.reshape(1, H1)
    ha, r1 = pl.pallas_call(
        _dense1_body,
        grid=(NB,),
        in_specs=[row_spec(NT), full(WeT), full(be_r), full(W1cat),
                  full(W1_root), full(b1_r)],
        out_specs=(row_spec(R * H1), row_spec(H1)),
        out_shape=(jax.ShapeDtypeStruct((N, R * H1), f32),
                   jax.ShapeDtypeStruct((N, H1), f32)),
    )(x, WeT, be_r, W1cat, W1_root, b1_r)

    tbl1 = ha.reshape(NROWS, H1)
    p1 = _sc_edge_call(tbl1, sflat3, dflat3, zeros_acc)
    p1r = p1[:, :NROWS].reshape(NCORE, N, R * H1)

    S = (jnp.arange(R * H1)[:, None] % H1 == jnp.arange(H1)[None, :]).astype(f32)
    W2a = jnp.transpose(W2[:, :, :H1], (1, 0, 2)).reshape(H1, R * H1)
    W2b = jnp.transpose(W2[:, :, H1:], (1, 0, 2)).reshape(H1, R * H1)
    b2_r = b2.reshape(1, H2)
    h2a, h2b, r2, cc = pl.pallas_call(
        _dense2_body,
        grid=(NB,),
        in_specs=[row_spec(R * H1), row_spec(R * H1), row_spec(R * H1),
                  row_spec(R * H1), row_spec(H1), full(S), full(W2a),
                  full(W2b), full(W2_root), full(b2_r)],
        out_specs=(row_spec(R * H1), row_spec(R * H1), row_spec(H2),
                   row_spec(R * H1)),
        out_shape=(jax.ShapeDtypeStruct((N, R * H1), f32),
                   jax.ShapeDtypeStruct((N, R * H1), f32),
                   jax.ShapeDtypeStruct((N, H2), f32),
                   jax.ShapeDtypeStruct((N, R * H1), f32)),
    )(cexp[0], cexp[1], p1r[0], p1r[1], r1, S, W2a, W2b, W2_root, b2_r)

    p2a = _sc_edge_call(h2a.reshape(NROWS, H1), sflat3, dflat3, zeros_acc)
    p2b = _sc_edge_call(h2b.reshape(NROWS, H1), sflat3, dflat3, zeros_acc)
    qa = p2a[:, :NROWS].reshape(NCORE, N, R * H1)
    qb = p2b[:, :NROWS].reshape(NCORE, N, R * H1)

    batch3 = batch.astype(jnp.int32).reshape(NB, 1, BN)
    sa, sb, cg = pl.pallas_call(
        _combine_pool_body,
        grid=(NB,),
        in_specs=[row_spec(R * H1), row_spec(R * H1), row_spec(R * H1),
                  row_spec(R * H1), row_spec(R * H1), row_spec(H2),
                  pl.BlockSpec((1, 1, BN), lambda i: (i, 0, 0)), full(S)],
        out_specs=(pl.BlockSpec((NG, H1), lambda i: (0, 0)),
                   pl.BlockSpec((NG, H1), lambda i: (0, 0)),
                   pl.BlockSpec((NG, H1), lambda i: (0, 0))),
        out_shape=(jax.ShapeDtypeStruct((NG, H1), f32),
                   jax.ShapeDtypeStruct((NG, H1), f32),
                   jax.ShapeDtypeStruct((NG, H1), f32)),
    )(qa[0], qa[1], qb[0], qb[1], cc, r2, batch3, S)

    fc1aT = fc1_W[:, :H1].T
    fc1bT = fc1_W[:, H1:].T
    fc1b_r = fc1_b.reshape(1, H1)
    fc2T = fc2_W.T
    fc2b_r = fc2_b.reshape(1, NC)
    out = pl.pallas_call(
        _head_body,
        out_shape=jax.ShapeDtypeStruct((NG, NC), f32),
    )(sa, sb, cg, fc1aT, fc1bT, fc1b_r, fc2T, fc2b_r)
    return out


# speedup vs baseline: 15.5816x; 1.0202x over previous
import functools
import jax
import jax.numpy as jnp
from jax import lax
from jax.experimental import pallas as pl
from jax.experimental.pallas import tpu as pltpu
from jax.experimental.pallas import tpu_sc as plsc

N = 10000
E = 320000
R = 8
NT = 8
F = 128
H1 = 16
H2 = 32
NG = 64
NC = 10

NCORE = 2
NSUB = 16
NW = NCORE * NSUB
LANE = 128
NBUF = 8
DEPTH = 4
RPT = 80
E_PAD = NW * LANE * RPT
NROWS = N * R
DUMMY = NROWS
ZCH = 5008
NROWS_PAD = ZCH * NSUB

BN = 1000
NB = N // BN


def _flat_idx_body(src_ref, dst_ref, typ_ref, sflat_ref, dflat_ref):
    t = typ_ref[...]
    sflat_ref[...] = src_ref[...] * R + t
    dflat_ref[...] = dst_ref[...] * R + t


def _dense1_body(x_ref, WeT_ref, be_ref, W1cat_ref, W1root_ref, b1_ref,
                 ha_ref, r1_ref):
    h = jnp.dot(x_ref[...], WeT_ref[...], preferred_element_type=jnp.float32)
    h = h + be_ref[...]
    ha_ref[...] = jnp.dot(h, W1cat_ref[...], preferred_element_type=jnp.float32)
    r1_ref[...] = jnp.dot(h, W1root_ref[...],
                          preferred_element_type=jnp.float32) + b1_ref[...]


def _dense2_body(c0_ref, c1_ref, p0_ref, p1_ref, r1_ref, S_ref,
                 W2a_ref, W2b_ref, W2root_ref, b2_ref,
                 h2a_ref, h2b_ref, r2_ref, cc_ref):
    cc = jnp.maximum(c0_ref[...] + c1_ref[...], 1.0)
    t = (p0_ref[...] + p1_ref[...]) / cc
    agg = jnp.dot(t, S_ref[...], preferred_element_type=jnp.float32)
    o1 = jnp.maximum(r1_ref[...] + agg, 0.0)
    h2a_ref[...] = jnp.dot(o1, W2a_ref[...], preferred_element_type=jnp.float32)
    h2b_ref[...] = jnp.dot(o1, W2b_ref[...], preferred_element_type=jnp.float32)
    r2_ref[...] = jnp.dot(o1, W2root_ref[...],
                          preferred_element_type=jnp.float32) + b2_ref[...]
    cc_ref[...] = cc


def _combine_pool_body(qa0_ref, qa1_ref, qb0_ref, qb1_ref, cc_ref, r2_ref,
                       batch_ref, S_ref, sa_ref, sb_ref, cg_ref):
    cc = cc_ref[...]
    ta = jnp.dot((qa0_ref[...] + qa1_ref[...]) / cc, S_ref[...],
                 preferred_element_type=jnp.float32)
    tb = jnp.dot((qb0_ref[...] + qb1_ref[...]) / cc, S_ref[...],
                 preferred_element_type=jnp.float32)
    r2 = r2_ref[...]
    o2a = jnp.maximum(r2[:, :H1] + ta, 0.0)
    o2b = jnp.maximum(r2[:, H1:] + tb, 0.0)
    brow = batch_ref[0]
    gids = lax.broadcasted_iota(jnp.int32, (NG, BN), 0)
    oh = (gids == jnp.broadcast_to(brow, (NG, BN))).astype(jnp.float32)
    sa_c = jnp.dot(oh, o2a, preferred_element_type=jnp.float32)
    sb_c = jnp.dot(oh, o2b, preferred_element_type=jnp.float32)
    cg_c = jnp.broadcast_to(jnp.sum(oh, axis=1, keepdims=True), (NG, H1))

    @pl.when(pl.program_id(0) == 0)
    def _init():
        sa_ref[...] = jnp.zeros_like(sa_ref)
        sb_ref[...] = jnp.zeros_like(sb_ref)
        cg_ref[...] = jnp.zeros_like(cg_ref)

    sa_ref[...] += sa_c
    sb_ref[...] += sb_c
    cg_ref[...] += cg_c


def _head_body(sa_ref, sb_ref, cg_ref, fc1aT_ref, fc1bT_ref, fc1b_ref,
               fc2T_ref, fc2b_ref, out_ref):
    c = jnp.maximum(cg_ref[...], 1.0)
    pa = sa_ref[...] / c
    pb = sb_ref[...] / c
    hh = jnp.dot(pa, fc1aT_ref[...], preferred_element_type=jnp.float32)
    hh = hh + jnp.dot(pb, fc1bT_ref[...], preferred_element_type=jnp.float32)
    hh = jnp.maximum(hh + fc1b_ref[...], 0.0)
    out_ref[...] = jnp.dot(hh, fc2T_ref[...],
                           preferred_element_type=jnp.float32) + fc2b_ref[...]


_SC_MESH = plsc.VectorSubcoreMesh(core_axis_name="c", subcore_axis_name="s")


def _zero_acc(zeros_hbm, acc, s):
    pltpu.sync_copy(zeros_hbm.at[pl.ds(s * ZCH, ZCH)],
                    acc.at[pl.ds(s * ZCH, ZCH)])


def _dump_acc(acc, out_hbm, c, s):
    plsc.subcore_barrier()
    pltpu.sync_copy(acc.at[pl.ds(s * ZCH, ZCH)],
                    out_hbm.at[c, pl.ds(s * ZCH, ZCH)])


def _cnt_phase(ones_v, dst_v, acc, sems):
    def group(jo, carry):
        for b in range(NBUF):
            @pl.when(jo > 0)
            def _w():
                pltpu.make_async_copy(ones_v, acc.at[dst_v.at[0]],
                                      sems[b]).wait()
            pltpu.async_copy(ones_v, acc.at[dst_v.at[jo * NBUF + b]],
                             sems[b], add=True)
        return carry

    lax.fori_loop(0, RPT // NBUF, group, 0)
    for b in range(NBUF):
        pltpu.make_async_copy(ones_v, acc.at[dst_v.at[0]], sems[b]).wait()


def _edge_phase(table_hbm, src_v, dst_v, msgs, acc, semg, sems):
    for b in range(DEPTH):
        pltpu.async_copy(table_hbm.at[src_v.at[b]], msgs.at[b], semg[b])

    def group(jo, carry):
        for b in range(NBUF):
            j = jo * NBUF + b
            pltpu.make_async_copy(table_hbm.at[src_v.at[0]], msgs.at[b],
                                  semg[b]).wait()
            pltpu.async_copy(msgs.at[b], acc.at[dst_v.at[j]], sems[b],
                             add=True)
            jn = j + DEPTH
            bn = (b + DEPTH) % NBUF

            @pl.when(jn < RPT)
            def _pf():
                @pl.when(j >= DEPTH)
                def _ws():
                    pltpu.make_async_copy(msgs.at[bn], acc.at[dst_v.at[0]],
                                          sems[bn]).wait()
                pltpu.async_copy(table_hbm.at[src_v.at[jn]], msgs.at[bn],
                                 semg[bn])
        return carry

    lax.fori_loop(0, RPT // NBUF, group, 0)
    for b in range(NBUF):
        pltpu.make_async_copy(msgs.at[b], acc.at[dst_v.at[0]], sems[b]).wait()


def _sc_cnt(dflat_hbm, ones_hbm, zeros_hbm, out_hbm,
            dst_v, ones_v, acc, sems):
    c = lax.axis_index("c")
    s = lax.axis_index("s")
    wid = c * NSUB + s
    _zero_acc(zeros_hbm, acc, s)
    pltpu.sync_copy(ones_hbm, ones_v)
    pltpu.sync_copy(dflat_hbm.at[wid], dst_v)
    plsc.subcore_barrier()
    _cnt_phase(ones_v, dst_v, acc, sems)
    _dump_acc(acc, out_hbm, c, s)


def _sc_edge(tbl_hbm, sflat_hbm, dflat_hbm, zeros_hbm, out_hbm,
             src_v, dst_v, msgs, acc, semg, sems):
    c = lax.axis_index("c")
    s = lax.axis_index("s")
    wid = c * NSUB + s
    _zero_acc(zeros_hbm, acc, s)
    pltpu.sync_copy(sflat_hbm.at[wid], src_v)
    pltpu.sync_copy(dflat_hbm.at[wid], dst_v)
    plsc.subcore_barrier()
    _edge_phase(tbl_hbm, src_v, dst_v, msgs, acc, semg, sems)
    _dump_acc(acc, out_hbm, c, s)


_SC_PARAMS = pltpu.CompilerParams(use_tc_tiling_on_sc=False)
_PART = jax.ShapeDtypeStruct((NCORE, NROWS_PAD, H1), jnp.float32)

_sc_cnt_call = pl.kernel(
    _sc_cnt,
    out_type=_PART,
    mesh=_SC_MESH,
    compiler_params=_SC_PARAMS,
    scratch_types=[
        pltpu.VMEM((RPT, LANE), jnp.int32),
        pltpu.VMEM((LANE, H1), jnp.float32),
        pltpu.VMEM_SHARED((NROWS_PAD, H1), jnp.float32),
        [pltpu.SemaphoreType.DMA] * NBUF,
    ],
)

_sc_edge_call = pl.kernel(
    _sc_edge,
    out_type=_PART,
    mesh=_SC_MESH,
    compiler_params=_SC_PARAMS,
    scratch_types=[
        pltpu.VMEM((RPT, LANE), jnp.int32),
        pltpu.VMEM((RPT, LANE), jnp.int32),
        pltpu.VMEM((NBUF, LANE, H1), jnp.float32),
        pltpu.VMEM_SHARED((NROWS_PAD, H1), jnp.float32),
        [pltpu.SemaphoreType.DMA] * NBUF,
        [pltpu.SemaphoreType.DMA] * NBUF,
    ],
)


def kernel(x, edge_index, edge_type, batch, We, be, W1, W1_root, b1,
           W2, W2_root, b2, fc1_W, fc1_b, fc2_W, fc2_b):
    f32 = jnp.float32
    pad = E_PAD - E
    src_p = jnp.pad(edge_index[0].astype(jnp.int32), (0, pad)).reshape(-1, LANE)
    dst_p = jnp.pad(edge_index[1].astype(jnp.int32), (0, pad),
                    constant_values=N).reshape(-1, LANE)
    typ_p = jnp.pad(edge_type.astype(jnp.int32), (0, pad)).reshape(-1, LANE)

    nrows2d = E_PAD // LANE
    sflat, dflat = pl.pallas_call(
        _flat_idx_body,
        out_shape=(jax.ShapeDtypeStruct((nrows2d, LANE), jnp.int32),
                   jax.ShapeDtypeStruct((nrows2d, LANE), jnp.int32)),
    )(src_p, dst_p, typ_p)
    sflat3 = sflat.reshape(NW, RPT, LANE)
    dflat3 = dflat.reshape(NW, RPT, LANE)

    zeros_acc = jnp.zeros((NROWS_PAD, H1), f32)
    ones_rows = jnp.ones((LANE, H1), f32)

    W1cat = jnp.transpose(W1, (1, 0, 2)).reshape(F, R * H1)
    row_spec = lambda w: pl.BlockSpec((BN, w), lambda i: (i, 0))
    full = lambda a: pl.BlockSpec(a.shape, lambda i: (0,) * a.ndim)
    WeT = We.T
    be_r = be.reshape(1, F)
    b1_r = b1.reshape(1, H1)
    ha, r1 = pl.pallas_call(
        _dense1_body,
        grid=(NB,),
        in_specs=[row_spec(NT), full(WeT), full(be_r), full(W1cat),
                  full(W1_root), full(b1_r)],
        out_specs=(row_spec(R * H1), row_spec(H1)),
        out_shape=(jax.ShapeDtypeStruct((N, R * H1), f32),
                   jax.ShapeDtypeStruct((N, H1), f32)),
    )(x, WeT, be_r, W1cat, W1_root, b1_r)

    cnt_part = _sc_cnt_call(dflat3, ones_rows, zeros_acc)
    cexp = cnt_part[:, :NROWS].reshape(NCORE, N, R * H1)
    tbl1 = ha.reshape(NROWS, H1)
    p1 = _sc_edge_call(tbl1, sflat3, dflat3, zeros_acc)
    p1r = p1[:, :NROWS].reshape(NCORE, N, R * H1)

    S = (jnp.arange(R * H1)[:, None] % H1 == jnp.arange(H1)[None, :]).astype(f32)
    W2a = jnp.transpose(W2[:, :, :H1], (1, 0, 2)).reshape(H1, R * H1)
    W2b = jnp.transpose(W2[:, :, H1:], (1, 0, 2)).reshape(H1, R * H1)
    b2_r = b2.reshape(1, H2)
    h2a, h2b, r2, cc = pl.pallas_call(
        _dense2_body,
        grid=(NB,),
        in_specs=[row_spec(R * H1), row_spec(R * H1), row_spec(R * H1),
                  row_spec(R * H1), row_spec(H1), full(S), full(W2a),
                  full(W2b), full(W2_root), full(b2_r)],
        out_specs=(row_spec(R * H1), row_spec(R * H1), row_spec(H2),
                   row_spec(R * H1)),
        out_shape=(jax.ShapeDtypeStruct((N, R * H1), f32),
                   jax.ShapeDtypeStruct((N, R * H1), f32),
                   jax.ShapeDtypeStruct((N, H2), f32),
                   jax.ShapeDtypeStruct((N, R * H1), f32)),
    )(cexp[0], cexp[1], p1r[0], p1r[1], r1, S, W2a, W2b, W2_root, b2_r)

    p2a = _sc_edge_call(h2a.reshape(NROWS, H1), sflat3, dflat3, zeros_acc)
    p2b = _sc_edge_call(h2b.reshape(NROWS, H1), sflat3, dflat3, zeros_acc)
    qa = p2a[:, :NROWS].reshape(NCORE, N, R * H1)
    qb = p2b[:, :NROWS].reshape(NCORE, N, R * H1)

    batch3 = batch.astype(jnp.int32).reshape(NB, 1, BN)
    sa, sb, cg = pl.pallas_call(
        _combine_pool_body,
        grid=(NB,),
        in_specs=[row_spec(R * H1), row_spec(R * H1), row_spec(R * H1),
                  row_spec(R * H1), row_spec(R * H1), row_spec(H2),
                  pl.BlockSpec((1, 1, BN), lambda i: (i, 0, 0)), full(S)],
        out_specs=(pl.BlockSpec((NG, H1), lambda i: (0, 0)),
                   pl.BlockSpec((NG, H1), lambda i: (0, 0)),
                   pl.BlockSpec((NG, H1), lambda i: (0, 0))),
        out_shape=(jax.ShapeDtypeStruct((NG, H1), f32),
                   jax.ShapeDtypeStruct((NG, H1), f32),
                   jax.ShapeDtypeStruct((NG, H1), f32)),
    )(qa[0], qa[1], qb[0], qb[1], cc, r2, batch3, S)

    fc1aT = fc1_W[:, :H1].T
    fc1bT = fc1_W[:, H1:].T
    fc1b_r = fc1_b.reshape(1, H1)
    fc2T = fc2_W.T
    fc2b_r = fc2_b.reshape(1, NC)
    out = pl.pallas_call(
        _head_body,
        out_shape=jax.ShapeDtypeStruct((NG, NC), f32),
    )(sa, sb, cg, fc1aT, fc1bT, fc1b_r, fc2T, fc2b_r)
    return out


# speedup vs baseline: 25.0150x; 1.6054x over previous
import functools
import jax
import jax.numpy as jnp
from jax import lax
from jax.experimental import pallas as pl
from jax.experimental.pallas import tpu as pltpu
from jax.experimental.pallas import tpu_sc as plsc

N = 10000
E = 320000
R = 8
NT = 8
F = 128
H1 = 16
H2 = 32
NG = 64
NC = 10

NCORE = 2
NSUB = 16
NW = NCORE * NSUB
LANE = 128
NBUF = 8
DEPTH = 4
RPT = 80
E_PAD = NW * LANE * RPT
NROWS = N * R
DUMMY = NROWS
ZCH = 5008
NROWS_PAD = ZCH * NSUB

BN = 1000
NB = N // BN


def _flat_idx_body(src_ref, dst_ref, typ_ref, sflat_ref, dflat_ref):
    t = typ_ref[...]
    sflat_ref[...] = src_ref[...] * R + t
    dflat_ref[...] = dst_ref[...] * R + t


def _dense1_body(x_ref, WeT_ref, be_ref, W1cat_ref, W1root_ref, b1_ref,
                 ha_ref, r1_ref):
    h = jnp.dot(x_ref[...], WeT_ref[...], preferred_element_type=jnp.float32)
    h = h + be_ref[...]
    ha_ref[...] = jnp.dot(h, W1cat_ref[...], preferred_element_type=jnp.float32)
    r1_ref[...] = jnp.dot(h, W1root_ref[...],
                          preferred_element_type=jnp.float32) + b1_ref[...]


def _dense2_body(c0_ref, c1_ref, p0_ref, p1_ref, r1_ref, S_ref,
                 W2a_ref, W2b_ref, W2root_ref, b2_ref,
                 h2a_ref, h2b_ref, r2_ref, cc_ref):
    cc = jnp.maximum(c0_ref[...] + c1_ref[...], 1.0)
    t = (p0_ref[...] + p1_ref[...]) / cc
    agg = jnp.dot(t, S_ref[...], preferred_element_type=jnp.float32)
    o1 = jnp.maximum(r1_ref[...] + agg, 0.0)
    h2a_ref[...] = jnp.dot(o1, W2a_ref[...], preferred_element_type=jnp.float32)
    h2b_ref[...] = jnp.dot(o1, W2b_ref[...], preferred_element_type=jnp.float32)
    r2_ref[...] = jnp.dot(o1, W2root_ref[...],
                          preferred_element_type=jnp.float32) + b2_ref[...]
    cc_ref[...] = cc


def _combine_pool_body(qa0_ref, qa1_ref, qb0_ref, qb1_ref, cc_ref, r2_ref,
                       batch_ref, S_ref, sa_ref, sb_ref, cg_ref):
    cc = cc_ref[...]
    ta = jnp.dot((qa0_ref[...] + qa1_ref[...]) / cc, S_ref[...],
                 preferred_element_type=jnp.float32)
    tb = jnp.dot((qb0_ref[...] + qb1_ref[...]) / cc, S_ref[...],
                 preferred_element_type=jnp.float32)
    r2 = r2_ref[...]
    o2a = jnp.maximum(r2[:, :H1] + ta, 0.0)
    o2b = jnp.maximum(r2[:, H1:] + tb, 0.0)
    brow = batch_ref[0]
    gids = lax.broadcasted_iota(jnp.int32, (NG, BN), 0)
    oh = (gids == jnp.broadcast_to(brow, (NG, BN))).astype(jnp.float32)
    sa_c = jnp.dot(oh, o2a, preferred_element_type=jnp.float32)
    sb_c = jnp.dot(oh, o2b, preferred_element_type=jnp.float32)
    cg_c = jnp.broadcast_to(jnp.sum(oh, axis=1, keepdims=True), (NG, H1))

    @pl.when(pl.program_id(0) == 0)
    def _init():
        sa_ref[...] = jnp.zeros_like(sa_ref)
        sb_ref[...] = jnp.zeros_like(sb_ref)
        cg_ref[...] = jnp.zeros_like(cg_ref)

    sa_ref[...] += sa_c
    sb_ref[...] += sb_c
    cg_ref[...] += cg_c


def _head_body(sa_ref, sb_ref, cg_ref, fc1aT_ref, fc1bT_ref, fc1b_ref,
               fc2T_ref, fc2b_ref, out_ref):
    c = jnp.maximum(cg_ref[...], 1.0)
    pa = sa_ref[...] / c
    pb = sb_ref[...] / c
    hh = jnp.dot(pa, fc1aT_ref[...], preferred_element_type=jnp.float32)
    hh = hh + jnp.dot(pb, fc1bT_ref[...], preferred_element_type=jnp.float32)
    hh = jnp.maximum(hh + fc1b_ref[...], 0.0)
    out_ref[...] = jnp.dot(hh, fc2T_ref[...],
                           preferred_element_type=jnp.float32) + fc2b_ref[...]


_SC_MESH = plsc.VectorSubcoreMesh(core_axis_name="c", subcore_axis_name="s")


def _zero_acc(zeros_hbm, acc, s):
    pltpu.sync_copy(zeros_hbm.at[pl.ds(s * ZCH, ZCH)],
                    acc.at[pl.ds(s * ZCH, ZCH)])


def _dump_acc(acc, out_hbm, c, s):
    plsc.subcore_barrier()
    pltpu.sync_copy(acc.at[pl.ds(s * ZCH, ZCH)],
                    out_hbm.at[c, pl.ds(s * ZCH, ZCH)])


def _cnt_phase(ones_v, dst_v, acc, sems):
    def group(jo, carry):
        for b in range(NBUF):
            @pl.when(jo > 0)
            def _w():
                pltpu.make_async_copy(ones_v, acc.at[dst_v.at[0]],
                                      sems[b]).wait()
            pltpu.async_copy(ones_v, acc.at[dst_v.at[jo * NBUF + b]],
                             sems[b], add=True)
        return carry

    lax.fori_loop(0, RPT // NBUF, group, 0)
    for b in range(NBUF):
        pltpu.make_async_copy(ones_v, acc.at[dst_v.at[0]], sems[b]).wait()


def _edge_phase(table_hbm, src_v, dst_v, msgs, acc, semg, sems):
    for b in range(DEPTH):
        pltpu.async_copy(table_hbm.at[src_v.at[b]], msgs.at[b], semg[b])

    def group(jo, carry):
        for b in range(NBUF):
            j = jo * NBUF + b
            pltpu.make_async_copy(table_hbm.at[src_v.at[0]], msgs.at[b],
                                  semg[b]).wait()
            pltpu.async_copy(msgs.at[b], acc.at[dst_v.at[j]], sems[b],
                             add=True)
            jn = j + DEPTH
            bn = (b + DEPTH) % NBUF

            @pl.when(jn < RPT)
            def _pf():
                @pl.when(j >= DEPTH)
                def _ws():
                    pltpu.make_async_copy(msgs.at[bn], acc.at[dst_v.at[0]],
                                          sems[bn]).wait()
                pltpu.async_copy(table_hbm.at[src_v.at[jn]], msgs.at[bn],
                                 semg[bn])
        return carry

    lax.fori_loop(0, RPT // NBUF, group, 0)
    for b in range(NBUF):
        pltpu.make_async_copy(msgs.at[b], acc.at[dst_v.at[0]], sems[b]).wait()


def _sc_cnt(dflat_hbm, ones_hbm, zeros_hbm, out_hbm,
            dst_v, ones_v, acc, sems):
    c = lax.axis_index("c")
    s = lax.axis_index("s")
    wid = c * NSUB + s
    _zero_acc(zeros_hbm, acc, s)
    pltpu.sync_copy(ones_hbm, ones_v)
    pltpu.sync_copy(dflat_hbm.at[wid], dst_v)
    plsc.subcore_barrier()
    _cnt_phase(ones_v, dst_v, acc, sems)
    _dump_acc(acc, out_hbm, c, s)


def _sc_edge(tbl_hbm, sflat_hbm, dflat_hbm, zeros_hbm, out_hbm,
             src_v, dst_v, msgs, acc, semg, sems):
    c = lax.axis_index("c")
    s = lax.axis_index("s")
    wid = c * NSUB + s
    _zero_acc(zeros_hbm, acc, s)
    pltpu.sync_copy(sflat_hbm.at[wid], src_v)
    pltpu.sync_copy(dflat_hbm.at[wid], dst_v)
    plsc.subcore_barrier()
    _edge_phase(tbl_hbm, src_v, dst_v, msgs, acc, semg, sems)
    _dump_acc(acc, out_hbm, c, s)


_SC_PARAMS = pltpu.CompilerParams(use_tc_tiling_on_sc=False)
_PART = jax.ShapeDtypeStruct((NCORE, NROWS_PAD, H1), jnp.float32)

_sc_cnt_call = pl.kernel(
    _sc_cnt,
    out_type=_PART,
    mesh=_SC_MESH,
    compiler_params=_SC_PARAMS,
    scratch_types=[
        pltpu.VMEM((RPT, LANE), jnp.int32),
        pltpu.VMEM((LANE, H1), jnp.float32),
        pltpu.VMEM_SHARED((NROWS_PAD, H1), jnp.float32),
        [pltpu.SemaphoreType.DMA] * NBUF,
    ],
)

_sc_edge_call = pl.kernel(
    _sc_edge,
    out_type=_PART,
    mesh=_SC_MESH,
    compiler_params=_SC_PARAMS,
    scratch_types=[
        pltpu.VMEM((RPT, LANE), jnp.int32),
        pltpu.VMEM((RPT, LANE), jnp.int32),
        pltpu.VMEM((NBUF, LANE, H1), jnp.float32),
        pltpu.VMEM_SHARED((NROWS_PAD, H1), jnp.float32),
        [pltpu.SemaphoreType.DMA] * NBUF,
        [pltpu.SemaphoreType.DMA] * NBUF,
    ],
)


def kernel(x, edge_index, edge_type, batch, We, be, W1, W1_root, b1,
           W2, W2_root, b2, fc1_W, fc1_b, fc2_W, fc2_b):
    f32 = jnp.float32
    pad = E_PAD - E
    src_p = jnp.pad(edge_index[0].astype(jnp.int32), (0, pad)).reshape(-1, LANE)
    dst_p = jnp.pad(edge_index[1].astype(jnp.int32), (0, pad),
                    constant_values=N).reshape(-1, LANE)
    typ_p = jnp.pad(edge_type.astype(jnp.int32), (0, pad)).reshape(-1, LANE)

    nrows2d = E_PAD // LANE
    sflat, dflat = pl.pallas_call(
        _flat_idx_body,
        out_shape=(jax.ShapeDtypeStruct((nrows2d, LANE), jnp.int32),
                   jax.ShapeDtypeStruct((nrows2d, LANE), jnp.int32)),
    )(src_p, dst_p, typ_p)
    sflat3 = sflat.reshape(NW, RPT, LANE)
    dflat3 = dflat.reshape(NW, RPT, LANE)

    zeros_acc = jnp.zeros((NROWS_PAD, H1), f32)
    ones_rows = jnp.ones((LANE, H1), f32)

    W1cat = jnp.transpose(W1, (1, 0, 2)).reshape(F, R * H1)
    row_spec = lambda w: pl.BlockSpec((BN, w), lambda i: (i, 0))
    full = lambda a: pl.BlockSpec(a.shape, lambda i: (0,) * a.ndim)
    WeT = We.T
    be_r = be.reshape(1, F)
    b1_r = b1.reshape(1, H1)
    ha, r1 = pl.pallas_call(
        _dense1_body,
        grid=(NB,),
        in_specs=[row_spec(NT), full(WeT), full(be_r), full(W1cat),
                  full(W1_root), full(b1_r)],
        out_specs=(row_spec(R * H1), row_spec(H1)),
        out_shape=(jax.ShapeDtypeStruct((N, R * H1), f32),
                   jax.ShapeDtypeStruct((N, H1), f32)),
    )(x, WeT, be_r, W1cat, W1_root, b1_r)

    NPR = NROWS_PAD // R
    cnt_part = _sc_cnt_call(dflat3, ones_rows, zeros_acc)
    cexp = cnt_part.reshape(NCORE, NPR, R * H1)
    tbl1 = ha.reshape(NROWS, H1)
    p1 = _sc_edge_call(tbl1, sflat3, dflat3, zeros_acc)
    p1r = p1.reshape(NCORE, NPR, R * H1)

    S = (jnp.arange(R * H1)[:, None] % H1 == jnp.arange(H1)[None, :]).astype(f32)
    W2a = jnp.transpose(W2[:, :, :H1], (1, 0, 2)).reshape(H1, R * H1)
    W2b = jnp.transpose(W2[:, :, H1:], (1, 0, 2)).reshape(H1, R * H1)
    b2_r = b2.reshape(1, H2)
    h2a, h2b, r2, cc = pl.pallas_call(
        _dense2_body,
        grid=(NB,),
        in_specs=[row_spec(R * H1), row_spec(R * H1), row_spec(R * H1),
                  row_spec(R * H1), row_spec(H1), full(S), full(W2a),
                  full(W2b), full(W2_root), full(b2_r)],
        out_specs=(row_spec(R * H1), row_spec(R * H1), row_spec(H2),
                   row_spec(R * H1)),
        out_shape=(jax.ShapeDtypeStruct((N, R * H1), f32),
                   jax.ShapeDtypeStruct((N, R * H1), f32),
                   jax.ShapeDtypeStruct((N, H2), f32),
                   jax.ShapeDtypeStruct((N, R * H1), f32)),
    )(cexp[0], cexp[1], p1r[0], p1r[1], r1, S, W2a, W2b, W2_root, b2_r)

    p2a = _sc_edge_call(h2a.reshape(NROWS, H1), sflat3, dflat3, zeros_acc)
    p2b = _sc_edge_call(h2b.reshape(NROWS, H1), sflat3, dflat3, zeros_acc)
    qa = p2a.reshape(NCORE, NPR, R * H1)
    qb = p2b.reshape(NCORE, NPR, R * H1)

    batch3 = batch.astype(jnp.int32).reshape(NB, 1, BN)
    sa, sb, cg = pl.pallas_call(
        _combine_pool_body,
        grid=(NB,),
        in_specs=[row_spec(R * H1), row_spec(R * H1), row_spec(R * H1),
                  row_spec(R * H1), row_spec(R * H1), row_spec(H2),
                  pl.BlockSpec((1, 1, BN), lambda i: (i, 0, 0)), full(S)],
        out_specs=(pl.BlockSpec((NG, H1), lambda i: (0, 0)),
                   pl.BlockSpec((NG, H1), lambda i: (0, 0)),
                   pl.BlockSpec((NG, H1), lambda i: (0, 0))),
        out_shape=(jax.ShapeDtypeStruct((NG, H1), f32),
                   jax.ShapeDtypeStruct((NG, H1), f32),
                   jax.ShapeDtypeStruct((NG, H1), f32)),
    )(qa[0], qa[1], qb[0], qb[1], cc, r2, batch3, S)

    fc1aT = fc1_W[:, :H1].T
    fc1bT = fc1_W[:, H1:].T
    fc1b_r = fc1_b.reshape(1, H1)
    fc2T = fc2_W.T
    fc2b_r = fc2_b.reshape(1, NC)
    out = pl.pallas_call(
        _head_body,
        out_shape=jax.ShapeDtypeStruct((NG, NC), f32),
    )(sa, sb, cg, fc1aT, fc1bT, fc1b_r, fc2T, fc2b_r)
    return out


# speedup vs baseline: 58.7351x; 2.3480x over previous
import functools
import jax
import jax.numpy as jnp
from jax import lax
from jax.experimental import pallas as pl
from jax.experimental.pallas import tpu as pltpu
from jax.experimental.pallas import tpu_sc as plsc

N = 10000
E = 320000
R = 8
NT = 8
F = 128
H1 = 16
H2 = 32
NG = 64
NC = 10

NCORE = 2
NSUB = 16
NW = NCORE * NSUB
LANE = 128
NBUF = 8
DEPTH = 4
RPT = 80
E_PAD = NW * LANE * RPT
NROWS = N * R
DUMMY = NROWS
ZCH = 5008
NROWS_PAD = ZCH * NSUB
NPR = NROWS_PAD // R
ZCH8 = ZCH // R

BN = 1000
NB = N // BN


def _flat_idx_body(src_ref, dst_ref, typ_ref, sflat_ref, dflat_ref):
    t = typ_ref[...]
    sflat_ref[...] = src_ref[...] * R + t
    dflat_ref[...] = dst_ref[...] * R + t


def _dense1_body(x_ref, WeT_ref, be_ref, W1cat_ref, W1root_ref, b1_ref,
                 ha_ref, r1_ref):
    h = jnp.dot(x_ref[...], WeT_ref[...], preferred_element_type=jnp.float32)
    h = h + be_ref[...]
    ha_ref[...] = jnp.dot(h, W1cat_ref[...], preferred_element_type=jnp.float32)
    r1_ref[...] = jnp.dot(h, W1root_ref[...],
                          preferred_element_type=jnp.float32) + b1_ref[...]


def _dense2_body(agg1_ref, r1_ref, W2a_ref, W2b_ref, W2root_ref, b2_ref,
                 h2a_ref, h2b_ref, r2_ref):
    o1 = jnp.maximum(r1_ref[...] + agg1_ref[...], 0.0)
    h2a_ref[...] = jnp.dot(o1, W2a_ref[...], preferred_element_type=jnp.float32)
    h2b_ref[...] = jnp.dot(o1, W2b_ref[...], preferred_element_type=jnp.float32)
    r2_ref[...] = jnp.dot(o1, W2root_ref[...],
                          preferred_element_type=jnp.float32) + b2_ref[...]


def _combine_pool_body(a2a_ref, a2b_ref, r2_ref,
                       batch_ref, sa_ref, sb_ref, cg_ref):
    r2 = r2_ref[...]
    o2a = jnp.maximum(r2[:, :H1] + a2a_ref[...], 0.0)
    o2b = jnp.maximum(r2[:, H1:] + a2b_ref[...], 0.0)
    brow = batch_ref[0]
    gids = lax.broadcasted_iota(jnp.int32, (NG, BN), 0)
    oh = (gids == jnp.broadcast_to(brow, (NG, BN))).astype(jnp.float32)
    sa_c = jnp.dot(oh, o2a, preferred_element_type=jnp.float32)
    sb_c = jnp.dot(oh, o2b, preferred_element_type=jnp.float32)
    cg_c = jnp.broadcast_to(jnp.sum(oh, axis=1, keepdims=True), (NG, H1))

    @pl.when(pl.program_id(0) == 0)
    def _init():
        sa_ref[...] = jnp.zeros_like(sa_ref)
        sb_ref[...] = jnp.zeros_like(sb_ref)
        cg_ref[...] = jnp.zeros_like(cg_ref)

    sa_ref[...] += sa_c
    sb_ref[...] += sb_c
    cg_ref[...] += cg_c


def _head_body(sa_ref, sb_ref, cg_ref, fc1aT_ref, fc1bT_ref, fc1b_ref,
               fc2T_ref, fc2b_ref, out_ref):
    c = jnp.maximum(cg_ref[...], 1.0)
    pa = sa_ref[...] / c
    pb = sb_ref[...] / c
    hh = jnp.dot(pa, fc1aT_ref[...], preferred_element_type=jnp.float32)
    hh = hh + jnp.dot(pb, fc1bT_ref[...], preferred_element_type=jnp.float32)
    hh = jnp.maximum(hh + fc1b_ref[...], 0.0)
    out_ref[...] = jnp.dot(hh, fc2T_ref[...],
                           preferred_element_type=jnp.float32) + fc2b_ref[...]


_SC_MESH = plsc.VectorSubcoreMesh(core_axis_name="c", subcore_axis_name="s")


def _zero_acc(zeros_hbm, acc, s):
    pltpu.sync_copy(zeros_hbm.at[pl.ds(s * ZCH, ZCH)],
                    acc.at[pl.ds(s * ZCH, ZCH)])


def _dump_acc(acc, out_hbm, c, s):
    plsc.subcore_barrier()
    pltpu.sync_copy(acc.at[pl.ds(s * ZCH, ZCH)],
                    out_hbm.at[c, pl.ds(s * ZCH, ZCH)])


def _cnt_phase(ones_v, dst_v, acc, sems):
    def group(jo, carry):
        for b in range(NBUF):
            @pl.when(jo > 0)
            def _w():
                pltpu.make_async_copy(ones_v, acc.at[dst_v.at[0]],
                                      sems[b]).wait()
            pltpu.async_copy(ones_v, acc.at[dst_v.at[jo * NBUF + b]],
                             sems[b], add=True)
        return carry

    lax.fori_loop(0, RPT // NBUF, group, 0)
    for b in range(NBUF):
        pltpu.make_async_copy(ones_v, acc.at[dst_v.at[0]], sems[b]).wait()


def _edge_phase(table_hbm, src_v, dst_v, msgs, acc, semg, sems):
    for b in range(DEPTH):
        pltpu.async_copy(table_hbm.at[src_v.at[b]], msgs.at[b], semg[b])

    def group(jo, carry):
        for b in range(NBUF):
            j = jo * NBUF + b
            pltpu.make_async_copy(table_hbm.at[src_v.at[0]], msgs.at[b],
                                  semg[b]).wait()
            pltpu.async_copy(msgs.at[b], acc.at[dst_v.at[j]], sems[b],
                             add=True)
            jn = j + DEPTH
            bn = (b + DEPTH) % NBUF

            @pl.when(jn < RPT)
            def _pf():
                @pl.when(j >= DEPTH)
                def _ws():
                    pltpu.make_async_copy(msgs.at[bn], acc.at[dst_v.at[0]],
                                          sems[bn]).wait()
                pltpu.async_copy(table_hbm.at[src_v.at[jn]], msgs.at[bn],
                                 semg[bn])
        return carry

    lax.fori_loop(0, RPT // NBUF, group, 0)
    for b in range(NBUF):
        pltpu.make_async_copy(msgs.at[b], acc.at[dst_v.at[0]], sems[b]).wait()


def _sc_cnt(dflat_hbm, ones_hbm, zeros_hbm, out_hbm,
            dst_v, ones_v, acc, sems):
    c = lax.axis_index("c")
    s = lax.axis_index("s")
    wid = c * NSUB + s
    _zero_acc(zeros_hbm, acc, s)
    pltpu.sync_copy(ones_hbm, ones_v)
    pltpu.sync_copy(dflat_hbm.at[wid], dst_v)
    plsc.subcore_barrier()
    _cnt_phase(ones_v, dst_v, acc, sems)
    _dump_acc(acc, out_hbm, c, s)


def _sc_edge(tbl_hbm, sflat_hbm, dflat_hbm, zeros_hbm, out_hbm,
             src_v, dst_v, msgs, acc, semg, sems):
    c = lax.axis_index("c")
    s = lax.axis_index("s")
    wid = c * NSUB + s
    _zero_acc(zeros_hbm, acc, s)
    pltpu.sync_copy(sflat_hbm.at[wid], src_v)
    pltpu.sync_copy(dflat_hbm.at[wid], dst_v)
    plsc.subcore_barrier()
    _edge_phase(tbl_hbm, src_v, dst_v, msgs, acc, semg, sems)
    _dump_acc(acc, out_hbm, c, s)


NPN = NPR // NW
CCH = 160
CST = NPN - CCH


def _comb_chunk(p_hbm, cnt_hbm, out_hbm, pa, pb, ca, cb, outv, rs):
    pltpu.sync_copy(p_hbm.at[0, pl.ds(rs * R, CCH * R)], pa)
    pltpu.sync_copy(p_hbm.at[1, pl.ds(rs * R, CCH * R)], pb)
    if cnt_hbm is not None:
        pltpu.sync_copy(cnt_hbm.at[0, pl.ds(rs * R, CCH * R)], ca)
        pltpu.sync_copy(cnt_hbm.at[1, pl.ds(rs * R, CCH * R)], cb)

    def node(j, carry):
        s = jnp.zeros((H1,), jnp.float32)
        for r in range(R):
            pp = pa[j * R + r, :] + pb[j * R + r, :]
            cc = jnp.maximum(ca[j * R + r, :] + cb[j * R + r, :], 1.0)
            s = s + pp / cc
        outv[j, :] = s
        return carry

    lax.fori_loop(0, CCH, node, 0)
    pltpu.sync_copy(outv, out_hbm.at[pl.ds(rs, CCH)])


def _sc_comb1(p_hbm, cnt_hbm, out_hbm, pa, pb, ca, cb, outv):
    wid = lax.axis_index("c") * NSUB + lax.axis_index("s")
    nb = wid * NPN
    _comb_chunk(p_hbm, cnt_hbm, out_hbm, pa, pb, ca, cb, outv, nb)
    _comb_chunk(p_hbm, cnt_hbm, out_hbm, pa, pb, ca, cb, outv, nb + CST)


def _sc_comb2(pa_hbm, pb_hbm, cnt_hbm, outa_hbm, outb_hbm,
              pa, pb, ca, cb, outv):
    wid = lax.axis_index("c") * NSUB + lax.axis_index("s")
    nb = wid * NPN
    for st in (nb, nb + CST):
        _comb_chunk(pa_hbm, cnt_hbm, outa_hbm, pa, pb, ca, cb, outv, st)
        _comb_chunk(pb_hbm, None, outb_hbm, pa, pb, ca, cb, outv, st)


_SC_PARAMS = pltpu.CompilerParams(use_tc_tiling_on_sc=False)
_PART = jax.ShapeDtypeStruct((NCORE, NROWS_PAD, H1), jnp.float32)
_AGG = jax.ShapeDtypeStruct((NPR, H1), jnp.float32)

_COMB_SCRATCH = [
    pltpu.VMEM((CCH * R, H1), jnp.float32),
    pltpu.VMEM((CCH * R, H1), jnp.float32),
    pltpu.VMEM((CCH * R, H1), jnp.float32),
    pltpu.VMEM((CCH * R, H1), jnp.float32),
    pltpu.VMEM((CCH, H1), jnp.float32),
]

_sc_comb1_call = pl.kernel(
    _sc_comb1,
    out_type=_AGG,
    mesh=_SC_MESH,
    compiler_params=_SC_PARAMS,
    scratch_types=_COMB_SCRATCH,
)

_sc_comb2_call = pl.kernel(
    _sc_comb2,
    out_type=(_AGG, _AGG),
    mesh=_SC_MESH,
    compiler_params=_SC_PARAMS,
    scratch_types=_COMB_SCRATCH,
)

_sc_cnt_call = pl.kernel(
    _sc_cnt,
    out_type=_PART,
    mesh=_SC_MESH,
    compiler_params=_SC_PARAMS,
    scratch_types=[
        pltpu.VMEM((RPT, LANE), jnp.int32),
        pltpu.VMEM((LANE, H1), jnp.float32),
        pltpu.VMEM_SHARED((NROWS_PAD, H1), jnp.float32),
        [pltpu.SemaphoreType.DMA] * NBUF,
    ],
)

_sc_edge_call = pl.kernel(
    _sc_edge,
    out_type=_PART,
    mesh=_SC_MESH,
    compiler_params=_SC_PARAMS,
    scratch_types=[
        pltpu.VMEM((RPT, LANE), jnp.int32),
        pltpu.VMEM((RPT, LANE), jnp.int32),
        pltpu.VMEM((NBUF, LANE, H1), jnp.float32),
        pltpu.VMEM_SHARED((NROWS_PAD, H1), jnp.float32),
        [pltpu.SemaphoreType.DMA] * NBUF,
        [pltpu.SemaphoreType.DMA] * NBUF,
    ],
)


def kernel(x, edge_index, edge_type, batch, We, be, W1, W1_root, b1,
           W2, W2_root, b2, fc1_W, fc1_b, fc2_W, fc2_b):
    f32 = jnp.float32
    pad = E_PAD - E
    ppos = jnp.arange(pad, dtype=jnp.int32)
    src_p = jnp.concatenate(
        [edge_index[0].astype(jnp.int32), ppos % N]).reshape(-1, LANE)
    dst_p = jnp.concatenate(
        [edge_index[1].astype(jnp.int32), N + (ppos % 16)]).reshape(-1, LANE)
    typ_p = jnp.concatenate(
        [edge_type.astype(jnp.int32), ppos % R]).reshape(-1, LANE)

    nrows2d = E_PAD // LANE
    sflat, dflat = pl.pallas_call(
        _flat_idx_body,
        out_shape=(jax.ShapeDtypeStruct((nrows2d, LANE), jnp.int32),
                   jax.ShapeDtypeStruct((nrows2d, LANE), jnp.int32)),
    )(src_p, dst_p, typ_p)
    sflat3 = sflat.reshape(NW, RPT, LANE)
    dflat3 = dflat.reshape(NW, RPT, LANE)

    zeros_acc = jnp.zeros((NROWS_PAD, H1), f32)
    ones_rows = jnp.ones((LANE, H1), f32)

    W1cat = jnp.transpose(W1, (1, 0, 2)).reshape(F, R * H1)
    row_spec = lambda w: pl.BlockSpec((BN, w), lambda i: (i, 0))
    full = lambda a: pl.BlockSpec(a.shape, lambda i: (0,) * a.ndim)
    WeT = We.T
    be_r = be.reshape(1, F)
    b1_r = b1.reshape(1, H1)
    ha, r1 = pl.pallas_call(
        _dense1_body,
        grid=(NB,),
        in_specs=[row_spec(NT), full(WeT), full(be_r), full(W1cat),
                  full(W1_root), full(b1_r)],
        out_specs=(row_spec(R * H1), row_spec(H1)),
        out_shape=(jax.ShapeDtypeStruct((N, R * H1), f32),
                   jax.ShapeDtypeStruct((N, H1), f32)),
    )(x, WeT, be_r, W1cat, W1_root, b1_r)

    cnt_part = _sc_cnt_call(dflat3, ones_rows, zeros_acc)
    p1 = _sc_edge_call(ha.reshape(NROWS, H1), sflat3, dflat3, zeros_acc)
    agg1 = _sc_comb1_call(p1, cnt_part)

    W2a = jnp.transpose(W2[:, :, :H1], (1, 0, 2)).reshape(H1, R * H1)
    W2b = jnp.transpose(W2[:, :, H1:], (1, 0, 2)).reshape(H1, R * H1)
    b2_r = b2.reshape(1, H2)
    h2a, h2b, r2 = pl.pallas_call(
        _dense2_body,
        grid=(NB,),
        in_specs=[row_spec(H1), row_spec(H1), full(W2a),
                  full(W2b), full(W2_root), full(b2_r)],
        out_specs=(row_spec(R * H1), row_spec(R * H1), row_spec(H2)),
        out_shape=(jax.ShapeDtypeStruct((N, R * H1), f32),
                   jax.ShapeDtypeStruct((N, R * H1), f32),
                   jax.ShapeDtypeStruct((N, H2), f32)),
    )(agg1, r1, W2a, W2b, W2_root, b2_r)

    p2a = _sc_edge_call(h2a.reshape(NROWS, H1), sflat3, dflat3, zeros_acc)
    p2b = _sc_edge_call(h2b.reshape(NROWS, H1), sflat3, dflat3, zeros_acc)
    agg2a, agg2b = _sc_comb2_call(p2a, p2b, cnt_part)

    batch3 = batch.astype(jnp.int32).reshape(NB, 1, BN)
    sa, sb, cg = pl.pallas_call(
        _combine_pool_body,
        grid=(NB,),
        in_specs=[row_spec(H1), row_spec(H1), row_spec(H2),
                  pl.BlockSpec((1, 1, BN), lambda i: (i, 0, 0))],
        out_specs=(pl.BlockSpec((NG, H1), lambda i: (0, 0)),
                   pl.BlockSpec((NG, H1), lambda i: (0, 0)),
                   pl.BlockSpec((NG, H1), lambda i: (0, 0))),
        out_shape=(jax.ShapeDtypeStruct((NG, H1), f32),
                   jax.ShapeDtypeStruct((NG, H1), f32),
                   jax.ShapeDtypeStruct((NG, H1), f32)),
    )(agg2a, agg2b, r2, batch3)

    fc1aT = fc1_W[:, :H1].T
    fc1bT = fc1_W[:, H1:].T
    fc1b_r = fc1_b.reshape(1, H1)
    fc2T = fc2_W.T
    fc2b_r = fc2_b.reshape(1, NC)
    out = pl.pallas_call(
        _head_body,
        out_shape=jax.ShapeDtypeStruct((NG, NC), f32),
    )(sa, sb, cg, fc1aT, fc1bT, fc1b_r, fc2T, fc2b_r)
    return out


# speedup vs baseline: 61.4015x; 1.0454x over previous
import functools
import jax
import jax.numpy as jnp
from jax import lax
from jax.experimental import pallas as pl
from jax.experimental.pallas import tpu as pltpu
from jax.experimental.pallas import tpu_sc as plsc

N = 10000
E = 320000
R = 8
NT = 8
F = 128
H1 = 16
H2 = 32
NG = 64
NC = 10

NCORE = 2
NSUB = 16
NW = NCORE * NSUB
LANE = 128
NBUF = 10
DEPTH = 5
RPT = 80
E_PAD = NW * LANE * RPT
NROWS = N * R
DUMMY = NROWS
ZCH = 5008
NROWS_PAD = ZCH * NSUB
NPR = NROWS_PAD // R
ZCH8 = ZCH // R

BN = 1000
NB = N // BN


def _flat_idx_body(src_ref, dst_ref, typ_ref, sflat_ref, dflat_ref):
    t = typ_ref[...]
    sflat_ref[...] = src_ref[...] * R + t
    dflat_ref[...] = dst_ref[...] * R + t


def _dense1_body(x_ref, WeT_ref, be_ref, W1cat_ref, W1root_ref, b1_ref,
                 ha_ref, r1_ref):
    h = jnp.dot(x_ref[...], WeT_ref[...], preferred_element_type=jnp.float32)
    h = h + be_ref[...]
    ha_ref[...] = jnp.dot(h, W1cat_ref[...], preferred_element_type=jnp.float32)
    r1_ref[...] = jnp.dot(h, W1root_ref[...],
                          preferred_element_type=jnp.float32) + b1_ref[...]


def _dense2_body(agg1_ref, r1_ref, W2a_ref, W2b_ref, W2root_ref, b2_ref,
                 h2a_ref, h2b_ref, r2_ref):
    o1 = jnp.maximum(r1_ref[...] + agg1_ref[...], 0.0)
    h2a_ref[...] = jnp.dot(o1, W2a_ref[...], preferred_element_type=jnp.float32)
    h2b_ref[...] = jnp.dot(o1, W2b_ref[...], preferred_element_type=jnp.float32)
    r2_ref[...] = jnp.dot(o1, W2root_ref[...],
                          preferred_element_type=jnp.float32) + b2_ref[...]


def _combine_pool_body(a2a_ref, a2b_ref, r2_ref,
                       batch_ref, sa_ref, sb_ref, cg_ref):
    r2 = r2_ref[...]
    o2a = jnp.maximum(r2[:, :H1] + a2a_ref[...], 0.0)
    o2b = jnp.maximum(r2[:, H1:] + a2b_ref[...], 0.0)
    brow = batch_ref[0]
    gids = lax.broadcasted_iota(jnp.int32, (NG, BN), 0)
    oh = (gids == jnp.broadcast_to(brow, (NG, BN))).astype(jnp.float32)
    sa_c = jnp.dot(oh, o2a, preferred_element_type=jnp.float32)
    sb_c = jnp.dot(oh, o2b, preferred_element_type=jnp.float32)
    cg_c = jnp.broadcast_to(jnp.sum(oh, axis=1, keepdims=True), (NG, H1))

    @pl.when(pl.program_id(0) == 0)
    def _init():
        sa_ref[...] = jnp.zeros_like(sa_ref)
        sb_ref[...] = jnp.zeros_like(sb_ref)
        cg_ref[...] = jnp.zeros_like(cg_ref)

    sa_ref[...] += sa_c
    sb_ref[...] += sb_c
    cg_ref[...] += cg_c


def _head_body(sa_ref, sb_ref, cg_ref, fc1aT_ref, fc1bT_ref, fc1b_ref,
               fc2T_ref, fc2b_ref, out_ref):
    c = jnp.maximum(cg_ref[...], 1.0)
    pa = sa_ref[...] / c
    pb = sb_ref[...] / c
    hh = jnp.dot(pa, fc1aT_ref[...], preferred_element_type=jnp.float32)
    hh = hh + jnp.dot(pb, fc1bT_ref[...], preferred_element_type=jnp.float32)
    hh = jnp.maximum(hh + fc1b_ref[...], 0.0)
    out_ref[...] = jnp.dot(hh, fc2T_ref[...],
                           preferred_element_type=jnp.float32) + fc2b_ref[...]


_SC_MESH = plsc.VectorSubcoreMesh(core_axis_name="c", subcore_axis_name="s")


def _zero_acc(zeros_hbm, acc, s):
    pltpu.sync_copy(zeros_hbm.at[pl.ds(s * ZCH, ZCH)],
                    acc.at[pl.ds(s * ZCH, ZCH)])


def _dump_acc(acc, out_hbm, c, s):
    plsc.subcore_barrier()
    pltpu.sync_copy(acc.at[pl.ds(s * ZCH, ZCH)],
                    out_hbm.at[c, pl.ds(s * ZCH, ZCH)])


def _cnt_phase(ones_v, dst_v, acc, sems):
    def group(jo, carry):
        for b in range(NBUF):
            @pl.when(jo > 0)
            def _w():
                pltpu.make_async_copy(ones_v, acc.at[dst_v.at[0]],
                                      sems[b]).wait()
            pltpu.async_copy(ones_v, acc.at[dst_v.at[jo * NBUF + b]],
                             sems[b], add=True)
        return carry

    lax.fori_loop(0, RPT // NBUF, group, 0)
    for b in range(NBUF):
        pltpu.make_async_copy(ones_v, acc.at[dst_v.at[0]], sems[b]).wait()


def _edge_phase(table_hbm, src_v, dst_v, msgs, acc, semg, sems):
    for b in range(DEPTH):
        pltpu.async_copy(table_hbm.at[src_v.at[b]], msgs.at[b], semg[b])

    def group(jo, carry):
        for b in range(NBUF):
            j = jo * NBUF + b
            pltpu.make_async_copy(table_hbm.at[src_v.at[0]], msgs.at[b],
                                  semg[b]).wait()
            pltpu.async_copy(msgs.at[b], acc.at[dst_v.at[j]], sems[b],
                             add=True)
            jn = j + DEPTH
            bn = (b + DEPTH) % NBUF

            @pl.when(jn < RPT)
            def _pf():
                @pl.when(j >= DEPTH)
                def _ws():
                    pltpu.make_async_copy(msgs.at[bn], acc.at[dst_v.at[0]],
                                          sems[bn]).wait()
                pltpu.async_copy(table_hbm.at[src_v.at[jn]], msgs.at[bn],
                                 semg[bn])
        return carry

    lax.fori_loop(0, RPT // NBUF, group, 0)
    for b in range(NBUF):
        pltpu.make_async_copy(msgs.at[b], acc.at[dst_v.at[0]], sems[b]).wait()


def _sc_cnt(dflat_hbm, ones_hbm, zeros_hbm, out_hbm,
            dst_v, ones_v, acc, sems):
    c = lax.axis_index("c")
    s = lax.axis_index("s")
    wid = c * NSUB + s
    _zero_acc(zeros_hbm, acc, s)
    pltpu.sync_copy(ones_hbm, ones_v)
    pltpu.sync_copy(dflat_hbm.at[wid], dst_v)
    plsc.subcore_barrier()
    _cnt_phase(ones_v, dst_v, acc, sems)
    _dump_acc(acc, out_hbm, c, s)


def _sc_edge(tbl_hbm, sflat_hbm, dflat_hbm, zeros_hbm, out_hbm,
             src_v, dst_v, msgs, acc, semg, sems):
    c = lax.axis_index("c")
    s = lax.axis_index("s")
    wid = c * NSUB + s
    _zero_acc(zeros_hbm, acc, s)
    pltpu.sync_copy(sflat_hbm.at[wid], src_v)
    pltpu.sync_copy(dflat_hbm.at[wid], dst_v)
    plsc.subcore_barrier()
    _edge_phase(tbl_hbm, src_v, dst_v, msgs, acc, semg, sems)
    _dump_acc(acc, out_hbm, c, s)


NPN = NPR // NW
CCH = 160
CST = NPN - CCH


def _comb_chunk(p_hbm, cnt_hbm, out_hbm, pa, pb, ca, cb, outv, rs):
    pltpu.sync_copy(p_hbm.at[0, pl.ds(rs * R, CCH * R)], pa)
    pltpu.sync_copy(p_hbm.at[1, pl.ds(rs * R, CCH * R)], pb)
    if cnt_hbm is not None:
        pltpu.sync_copy(cnt_hbm.at[0, pl.ds(rs * R, CCH * R)], ca)
        pltpu.sync_copy(cnt_hbm.at[1, pl.ds(rs * R, CCH * R)], cb)

    def node(j, carry):
        s = jnp.zeros((H1,), jnp.float32)
        for r in range(R):
            pp = pa[j * R + r, :] + pb[j * R + r, :]
            cc = jnp.maximum(ca[j * R + r, :] + cb[j * R + r, :], 1.0)
            s = s + pp / cc
        outv[j, :] = s
        return carry

    lax.fori_loop(0, CCH, node, 0)
    pltpu.sync_copy(outv, out_hbm.at[pl.ds(rs, CCH)])


def _sc_comb1(p_hbm, cnt_hbm, out_hbm, pa, pb, ca, cb, outv):
    wid = lax.axis_index("c") * NSUB + lax.axis_index("s")
    nb = wid * NPN
    _comb_chunk(p_hbm, cnt_hbm, out_hbm, pa, pb, ca, cb, outv, nb)
    _comb_chunk(p_hbm, cnt_hbm, out_hbm, pa, pb, ca, cb, outv, nb + CST)


def _sc_comb2(pa_hbm, pb_hbm, cnt_hbm, outa_hbm, outb_hbm,
              pa0, pa1, pb0, pb1, ca, cb, outa, outb):
    wid = lax.axis_index("c") * NSUB + lax.axis_index("s")
    nb = wid * NPN
    for st in (nb, nb + CST):
        rs = st * R
        pltpu.sync_copy(pa_hbm.at[0, pl.ds(rs, CCH * R)], pa0)
        pltpu.sync_copy(pa_hbm.at[1, pl.ds(rs, CCH * R)], pa1)
        pltpu.sync_copy(pb_hbm.at[0, pl.ds(rs, CCH * R)], pb0)
        pltpu.sync_copy(pb_hbm.at[1, pl.ds(rs, CCH * R)], pb1)
        pltpu.sync_copy(cnt_hbm.at[0, pl.ds(rs, CCH * R)], ca)
        pltpu.sync_copy(cnt_hbm.at[1, pl.ds(rs, CCH * R)], cb)

        def node(j, carry):
            sa_v = jnp.zeros((H1,), jnp.float32)
            sb_v = jnp.zeros((H1,), jnp.float32)
            for r in range(R):
                k = j * R + r
                riv = 1.0 / jnp.maximum(ca[k, :] + cb[k, :], 1.0)
                sa_v = sa_v + (pa0[k, :] + pa1[k, :]) * riv
                sb_v = sb_v + (pb0[k, :] + pb1[k, :]) * riv
            outa[j, :] = sa_v
            outb[j, :] = sb_v
            return carry

        lax.fori_loop(0, CCH, node, 0)
        pltpu.sync_copy(outa, outa_hbm.at[pl.ds(st, CCH)])
        pltpu.sync_copy(outb, outb_hbm.at[pl.ds(st, CCH)])


_SC_PARAMS = pltpu.CompilerParams(use_tc_tiling_on_sc=False)
_PART = jax.ShapeDtypeStruct((NCORE, NROWS_PAD, H1), jnp.float32)
_AGG = jax.ShapeDtypeStruct((NPR, H1), jnp.float32)

_COMB_SCRATCH = [
    pltpu.VMEM((CCH * R, H1), jnp.float32),
    pltpu.VMEM((CCH * R, H1), jnp.float32),
    pltpu.VMEM((CCH * R, H1), jnp.float32),
    pltpu.VMEM((CCH * R, H1), jnp.float32),
    pltpu.VMEM((CCH, H1), jnp.float32),
]

_sc_comb1_call = pl.kernel(
    _sc_comb1,
    out_type=_AGG,
    mesh=_SC_MESH,
    compiler_params=_SC_PARAMS,
    scratch_types=_COMB_SCRATCH,
)

_sc_comb2_call = pl.kernel(
    _sc_comb2,
    out_type=(_AGG, _AGG),
    mesh=_SC_MESH,
    compiler_params=_SC_PARAMS,
    scratch_types=[
        pltpu.VMEM((CCH * R, H1), jnp.float32),
        pltpu.VMEM((CCH * R, H1), jnp.float32),
        pltpu.VMEM((CCH * R, H1), jnp.float32),
        pltpu.VMEM((CCH * R, H1), jnp.float32),
        pltpu.VMEM((CCH * R, H1), jnp.float32),
        pltpu.VMEM((CCH * R, H1), jnp.float32),
        pltpu.VMEM((CCH, H1), jnp.float32),
        pltpu.VMEM((CCH, H1), jnp.float32),
    ],
)

_sc_cnt_call = pl.kernel(
    _sc_cnt,
    out_type=_PART,
    mesh=_SC_MESH,
    compiler_params=_SC_PARAMS,
    scratch_types=[
        pltpu.VMEM((RPT, LANE), jnp.int32),
        pltpu.VMEM((LANE, H1), jnp.float32),
        pltpu.VMEM_SHARED((NROWS_PAD, H1), jnp.float32),
        [pltpu.SemaphoreType.DMA] * NBUF,
    ],
)

_sc_edge_call = pl.kernel(
    _sc_edge,
    out_type=_PART,
    mesh=_SC_MESH,
    compiler_params=_SC_PARAMS,
    scratch_types=[
        pltpu.VMEM((RPT, LANE), jnp.int32),
        pltpu.VMEM((RPT, LANE), jnp.int32),
        pltpu.VMEM((NBUF, LANE, H1), jnp.float32),
        pltpu.VMEM_SHARED((NROWS_PAD, H1), jnp.float32),
        [pltpu.SemaphoreType.DMA] * NBUF,
        [pltpu.SemaphoreType.DMA] * NBUF,
    ],
)


def kernel(x, edge_index, edge_type, batch, We, be, W1, W1_root, b1,
           W2, W2_root, b2, fc1_W, fc1_b, fc2_W, fc2_b):
    f32 = jnp.float32
    pad = E_PAD - E
    ppos = jnp.arange(pad, dtype=jnp.int32)
    src_p = jnp.concatenate(
        [edge_index[0].astype(jnp.int32), ppos % N]).reshape(-1, LANE)
    dst_p = jnp.concatenate(
        [edge_index[1].astype(jnp.int32), N + (ppos % 16)]).reshape(-1, LANE)
    typ_p = jnp.concatenate(
        [edge_type.astype(jnp.int32), ppos % R]).reshape(-1, LANE)

    nrows2d = E_PAD // LANE
    sflat, dflat = pl.pallas_call(
        _flat_idx_body,
        out_shape=(jax.ShapeDtypeStruct((nrows2d, LANE), jnp.int32),
                   jax.ShapeDtypeStruct((nrows2d, LANE), jnp.int32)),
    )(src_p, dst_p, typ_p)
    sflat3 = sflat.reshape(NW, RPT, LANE)
    dflat3 = dflat.reshape(NW, RPT, LANE)

    zeros_acc = jnp.zeros((NROWS_PAD, H1), f32)
    ones_rows = jnp.ones((LANE, H1), f32)

    W1cat = jnp.transpose(W1, (1, 0, 2)).reshape(F, R * H1)
    row_spec = lambda w: pl.BlockSpec((BN, w), lambda i: (i, 0))
    full = lambda a: pl.BlockSpec(a.shape, lambda i: (0,) * a.ndim)
    WeT = We.T
    be_r = be.reshape(1, F)
    b1_r = b1.reshape(1, H1)
    ha, r1 = pl.pallas_call(
        _dense1_body,
        grid=(NB,),
        in_specs=[row_spec(NT), full(WeT), full(be_r), full(W1cat),
                  full(W1_root), full(b1_r)],
        out_specs=(row_spec(R * H1), row_spec(H1)),
        out_shape=(jax.ShapeDtypeStruct((N, R * H1), f32),
                   jax.ShapeDtypeStruct((N, H1), f32)),
    )(x, WeT, be_r, W1cat, W1_root, b1_r)

    cnt_part = _sc_cnt_call(dflat3, ones_rows, zeros_acc)
    p1 = _sc_edge_call(ha.reshape(NROWS, H1), sflat3, dflat3, zeros_acc)
    agg1 = _sc_comb1_call(p1, cnt_part)

    W2a = jnp.transpose(W2[:, :, :H1], (1, 0, 2)).reshape(H1, R * H1)
    W2b = jnp.transpose(W2[:, :, H1:], (1, 0, 2)).reshape(H1, R * H1)
    b2_r = b2.reshape(1, H2)
    h2a, h2b, r2 = pl.pallas_call(
        _dense2_body,
        grid=(NB,),
        in_specs=[row_spec(H1), row_spec(H1), full(W2a),
                  full(W2b), full(W2_root), full(b2_r)],
        out_specs=(row_spec(R * H1), row_spec(R * H1), row_spec(H2)),
        out_shape=(jax.ShapeDtypeStruct((N, R * H1), f32),
                   jax.ShapeDtypeStruct((N, R * H1), f32),
                   jax.ShapeDtypeStruct((N, H2), f32)),
    )(agg1, r1, W2a, W2b, W2_root, b2_r)

    p2a = _sc_edge_call(h2a.reshape(NROWS, H1), sflat3, dflat3, zeros_acc)
    p2b = _sc_edge_call(h2b.reshape(NROWS, H1), sflat3, dflat3, zeros_acc)
    agg2a, agg2b = _sc_comb2_call(p2a, p2b, cnt_part)

    batch3 = batch.astype(jnp.int32).reshape(NB, 1, BN)
    sa, sb, cg = pl.pallas_call(
        _combine_pool_body,
        grid=(NB,),
        in_specs=[row_spec(H1), row_spec(H1), row_spec(H2),
                  pl.BlockSpec((1, 1, BN), lambda i: (i, 0, 0))],
        out_specs=(pl.BlockSpec((NG, H1), lambda i: (0, 0)),
                   pl.BlockSpec((NG, H1), lambda i: (0, 0)),
                   pl.BlockSpec((NG, H1), lambda i: (0, 0))),
        out_shape=(jax.ShapeDtypeStruct((NG, H1), f32),
                   jax.ShapeDtypeStruct((NG, H1), f32),
                   jax.ShapeDtypeStruct((NG, H1), f32)),
    )(agg2a, agg2b, r2, batch3)

    fc1aT = fc1_W[:, :H1].T
    fc1bT = fc1_W[:, H1:].T
    fc1b_r = fc1_b.reshape(1, H1)
    fc2T = fc2_W.T
    fc2b_r = fc2_b.reshape(1, NC)
    out = pl.pallas_call(
        _head_body,
        out_shape=jax.ShapeDtypeStruct((NG, NC), f32),
    )(sa, sb, cg, fc1aT, fc1bT, fc1b_r, fc2T, fc2b_r)
    return out
